# baseline (speedup 1.0000x reference)
import functools

import jax
import jax.numpy as jnp
from jax.experimental import pallas as pl
from jax.experimental.pallas import tpu as pltpu

_B = 16384
_D = 64
_K = 1000
_BS = 2048
_NB = _B // _BS
_HI = 32

_INV_SQRT_2PI = 0.3989422804014327

_CONTRACT0 = (((0,), (0,)), ((), ()))
_CONTRACT_NT = (((1,), (1,)), ((), ()))
_PREC = jax.lax.Precision.HIGHEST


def _cluster_kernel(feature_ref, cnt_ref, label_ref, weight_ref, cosf_ref):
    eps = 1e-8
    cnt = cnt_ref[...]
    cnorm = jnp.sqrt(jnp.sum(cnt * cnt, axis=1, keepdims=True))
    cnn = cnt / jnp.maximum(cnorm, eps)

    col_iota = jax.lax.broadcasted_iota(jnp.int32, (_BS, _K), 1)
    iota32 = jax.lax.broadcasted_iota(jnp.int32, (_BS, _HI), 1)

    def onehots(lab):
        oh_hi = (iota32 == (lab >> 5)).astype(jnp.float32)
        oh_lo = (iota32 == (lab & 31)).astype(jnp.float32)
        return oh_hi, oh_lo

    def phase1(j, acc):
        f = feature_ref[pl.ds(j * _BS, _BS), :]
        fnorm = jnp.sqrt(jnp.sum(f * f, axis=1, keepdims=True))
        fn = f / jnp.maximum(fnorm, eps)
        cos = jax.lax.dot_general(
            fn, cnn, _CONTRACT_NT,
            preferred_element_type=jnp.float32)
        cos_f = jnp.max(cos, axis=1, keepdims=True)
        lab = jnp.argmax(cos, axis=1).astype(jnp.int32)[:, None]
        oh_hi, oh_lo = onehots(lab)
        ohv = oh_hi * cos_f
        lhs = jnp.concatenate([oh_hi, ohv, ohv * cos_f], axis=1)
        acc = acc + jax.lax.dot_general(
            lhs, oh_lo, _CONTRACT0, preferred_element_type=jnp.float32,
            precision=_PREC)
        cosf_ref[pl.ds(j * _BS, _BS), :] = cos_f
        label_ref[pl.ds(j * _BS, _BS), :] = lab.astype(jnp.float32)
        return acc

    acc = jax.lax.fori_loop(
        0, _NB, phase1, jnp.zeros((3 * _HI, _HI), jnp.float32))
    counts, sums, sumsq = acc[:_HI], acc[_HI:2 * _HI], acc[2 * _HI:]
    mean = sums / jnp.maximum(counts, 1.0)
    sq = sumsq - counts * mean * mean
    var = sq / jnp.maximum(counts - 1.0, 1.0)
    inv_std = jax.lax.rsqrt(jnp.maximum(var, 1e-12))
    stats = jnp.concatenate([mean, inv_std], axis=1)

    def phase2(j, _):
        cos_f = cosf_ref[pl.ds(j * _BS, _BS), :]
        lab = label_ref[pl.ds(j * _BS, _BS), :].astype(jnp.int32)
        oh_hi, oh_lo = onehots(lab)
        rows = jnp.dot(oh_hi, stats, preferred_element_type=jnp.float32,
                       precision=_PREC)
        mean_g = jnp.sum(rows[:, :_HI] * oh_lo, axis=1, keepdims=True)
        isd_g = jnp.sum(rows[:, _HI:] * oh_lo, axis=1, keepdims=True)
        z = (cos_f - mean_g) * isd_g
        pdf = jnp.exp(-0.5 * z * z) * isd_g * _INV_SQRT_2PI
        w = jnp.where(cos_f < mean_g, pdf, 1.0)
        weight_ref[pl.ds(j * _BS, _BS), :] = w
        return 0

    jax.lax.fori_loop(0, _NB, phase2, 0)


@functools.partial(jax.jit, static_argnames=())
def kernel(feature, pred, unlabeled_index, centroids):
    del pred, unlabeled_index
    label2d, weight2d = pl.pallas_call(
        _cluster_kernel,
        out_shape=(
            jax.ShapeDtypeStruct((_B, 1), jnp.float32),
            jax.ShapeDtypeStruct((_B, 1), jnp.float32),
        ),
        scratch_shapes=[
            pltpu.VMEM((_B, 1), jnp.float32),
        ],
    )(feature, centroids)
    return label2d.reshape(_B), weight2d.reshape(_B)


# speedup vs baseline: 1.2986x; 1.2986x over previous
import functools

import jax
import jax.numpy as jnp
from jax.experimental import pallas as pl
from jax.experimental.pallas import tpu as pltpu
from jax.experimental.pallas import tpu_sc as plsc

_B = 16384
_D = 64
_K = 1000
_BS = 2048
_NB = _B // _BS

_NSUB = 16
_CH = _B // _NSUB
_KS = 1024
_BINS = 3 * _KS
_SL = _BINS // _NSUB

_INV_SQRT_2PI = 0.3989422804014327
_CONTRACT_NT = (((1,), (1,)), ((), ()))


def _tc_kernel(feature_ref, cnt_ref, label_ref, cosf_ref):
    eps = 1e-8
    cnt = cnt_ref[...]
    cnorm = jnp.sqrt(jnp.sum(cnt * cnt, axis=1, keepdims=True))
    cnn = cnt / jnp.maximum(cnorm, eps)

    def body(j, _):
        f = feature_ref[pl.ds(j * _BS, _BS), :]
        fnorm = jnp.sqrt(jnp.sum(f * f, axis=1, keepdims=True))
        fn = f / jnp.maximum(fnorm, eps)
        cos = jax.lax.dot_general(fn, cnn, _CONTRACT_NT,
                                  preferred_element_type=jnp.float32)
        cos_f = jnp.max(cos, axis=1, keepdims=True)
        col_iota = jax.lax.broadcasted_iota(jnp.int32, (_BS, _K), 1)
        lab = jnp.min(jnp.where(cos == cos_f, col_iota, _K), axis=1,
                      keepdims=True)
        cosf_ref[pl.ds(j * _BS, _BS), :] = cos_f
        label_ref[pl.ds(j * _BS, _BS), :] = lab.astype(jnp.float32)
        return 0

    jax.lax.fori_loop(0, _NB, body, 0)


def _sc_weight(cosf_hbm, lab_hbm, w_hbm,
               cf_v, lab_v, bins_v, mean_v, isd_v, out_v, acc_v,
               stage, total):
    sid = jax.lax.axis_index("s")
    base = sid * _CH
    pltpu.sync_copy(cosf_hbm.at[pl.ds(base, _CH)], cf_v)
    pltpu.sync_copy(lab_hbm.at[pl.ds(base, _CH)], lab_v)

    zeros16 = jnp.zeros((16,), jnp.float32)
    ones16 = jnp.ones((16,), jnp.float32)

    def zbody(i, _):
        bins_v[pl.ds(i * 16, 16)] = zeros16
        return 0

    jax.lax.fori_loop(0, _BINS // 16, zbody, 0)

    koff = jnp.full((16,), _KS, jnp.int32)

    def sbody(i, _):
        sl = pl.ds(i * 16, 16)
        lab16 = lab_v[sl].astype(jnp.int32)
        cf16 = cf_v[sl]
        plsc.addupdate_scatter(bins_v, [lab16], ones16)
        plsc.addupdate_scatter(bins_v, [lab16 + koff], cf16)
        plsc.addupdate_scatter(bins_v, [lab16 + koff + koff], cf16 * cf16)
        return 0

    jax.lax.fori_loop(0, _CH // 16, sbody, 0)

    pltpu.sync_copy(bins_v, stage.at[pl.ds(sid * _BINS, _BINS)])
    plsc.subcore_barrier()
    off = sid * _SL
    for r in range(_NSUB):
        pltpu.sync_copy(stage.at[pl.ds(r * _BINS + off, _SL)],
                        acc_v.at[pl.ds(r * _SL, _SL)])
    for i in range(_SL // 16):
        sl = pl.ds(i * 16, 16)
        tot = acc_v[sl]
        for r in range(1, _NSUB):
            tot = tot + acc_v[pl.ds(r * _SL + i * 16, 16)]
        bins_v[pl.ds(i * 16, 16)] = tot
    pltpu.sync_copy(bins_v.at[pl.ds(0, _SL)], total.at[pl.ds(off, _SL)])
    plsc.subcore_barrier()
    pltpu.sync_copy(total, bins_v)

    def mbody(i, _):
        sl = pl.ds(i * 16, 16)
        c = bins_v[sl]
        s = bins_v[pl.ds(_KS + i * 16, 16)]
        q = bins_v[pl.ds(2 * _KS + i * 16, 16)]
        m = s / jnp.maximum(c, 1.0)
        var = (q - c * m * m) / jnp.maximum(c - 1.0, 1.0)
        var = jnp.maximum(var, 1e-12)
        ib = plsc.bitcast(var, jnp.int32)
        y = plsc.bitcast(jnp.int32(0x5F3759DF) - (ib >> 1), jnp.float32)
        y = y * (1.5 - 0.5 * var * y * y)
        y = y * (1.5 - 0.5 * var * y * y)
        y = y * (1.5 - 0.5 * var * y * y)
        mean_v[sl] = m
        isd_v[sl] = y
        return 0

    jax.lax.fori_loop(0, _KS // 16, mbody, 0)

    def wbody(i, _):
        sl = pl.ds(i * 16, 16)
        lab16 = lab_v[sl].astype(jnp.int32)
        cf16 = cf_v[sl]
        m = plsc.load_gather(mean_v, [lab16])
        isd = plsc.load_gather(isd_v, [lab16])
        z = (cf16 - m) * isd
        pdf = jnp.exp(-0.5 * z * z) * isd * _INV_SQRT_2PI
        out_v[sl] = jnp.where(cf16 < m, pdf, ones16)
        return 0

    jax.lax.fori_loop(0, _CH // 16, wbody, 0)
    pltpu.sync_copy(out_v, w_hbm.at[pl.ds(base, _CH)])


_sc_call = functools.partial(
    pl.kernel,
    mesh=plsc.VectorSubcoreMesh(core_axis_name="c", subcore_axis_name="s",
                                num_cores=1),
    compiler_params=pltpu.CompilerParams(needs_layout_passes=False),
    out_type=jax.ShapeDtypeStruct((_B,), jnp.float32),
    scratch_types=[
        pltpu.VMEM((_CH,), jnp.float32),
        pltpu.VMEM((_CH,), jnp.float32),
        pltpu.VMEM((_BINS,), jnp.float32),
        pltpu.VMEM((_KS,), jnp.float32),
        pltpu.VMEM((_KS,), jnp.float32),
        pltpu.VMEM((_CH,), jnp.float32),
        pltpu.VMEM((_NSUB * _SL,), jnp.float32),
        pltpu.VMEM_SHARED((_NSUB * _BINS,), jnp.float32),
        pltpu.VMEM_SHARED((_BINS,), jnp.float32),
    ],
)(_sc_weight)


@functools.partial(jax.jit, static_argnames=())
def kernel(feature, pred, unlabeled_index, centroids):
    del pred, unlabeled_index
    label2d, cosf2d = pl.pallas_call(
        _tc_kernel,
        out_shape=(
            jax.ShapeDtypeStruct((_B, 1), jnp.float32),
            jax.ShapeDtypeStruct((_B, 1), jnp.float32),
        ),
    )(feature, centroids)
    label = label2d.reshape(_B)
    cosf = cosf2d.reshape(_B)
    weight = _sc_call(cosf, label)
    return label, weight


# speedup vs baseline: 1.3285x; 1.0230x over previous
import functools

import jax
import jax.numpy as jnp
from jax.experimental import pallas as pl
from jax.experimental.pallas import tpu as pltpu
from jax.experimental.pallas import tpu_sc as plsc

_B = 16384
_D = 64
_K = 1000
_BS = 4096
_NB = _B // _BS

_NSUB = 16
_CH = _B // _NSUB
_KS = 1024
_BINS = 3 * _KS
_SL = _BINS // _NSUB

_INV_SQRT_2PI = 0.3989422804014327
_CONTRACT_NT = (((1,), (1,)), ((), ()))


def _tc_kernel(feature_ref, cnt_ref, label_ref, cosf_ref):
    eps = 1e-8
    cnt = cnt_ref[...]
    cnorm = jnp.sqrt(jnp.sum(cnt * cnt, axis=1, keepdims=True))
    cnn = cnt / jnp.maximum(cnorm, eps)

    def body(j, _):
        f = feature_ref[pl.ds(j * _BS, _BS), :]
        fnorm = jnp.sqrt(jnp.sum(f * f, axis=1, keepdims=True))
        fn = f / jnp.maximum(fnorm, eps)
        cos = jax.lax.dot_general(fn, cnn, _CONTRACT_NT,
                                  preferred_element_type=jnp.float32)
        cos_f = jnp.max(cos, axis=1, keepdims=True)
        col_iota = jax.lax.broadcasted_iota(jnp.int32, (_BS, _K), 1)
        lab = jnp.min(jnp.where(cos == cos_f, col_iota, _K), axis=1,
                      keepdims=True)
        cosf_ref[pl.ds(j * _BS, _BS), :] = cos_f
        label_ref[pl.ds(j * _BS, _BS), :] = lab.astype(jnp.float32)
        return 0

    jax.lax.fori_loop(0, _NB, body, 0)


def _sc_weight(cosf_hbm, lab_hbm, w_hbm,
               cf_v, lab_v, bins_v, mean_v, isd_v, out_v, acc_v,
               stage, total):
    sid = jax.lax.axis_index("s")
    base = sid * _CH
    pltpu.sync_copy(cosf_hbm.at[pl.ds(base, _CH)], cf_v)
    pltpu.sync_copy(lab_hbm.at[pl.ds(base, _CH)], lab_v)

    zeros16 = jnp.zeros((16,), jnp.float32)
    ones16 = jnp.ones((16,), jnp.float32)

    def zbody(i, _):
        bins_v[pl.ds(i * 16, 16)] = zeros16
        return 0

    jax.lax.fori_loop(0, _BINS // 16, zbody, 0)

    koff = jnp.full((16,), _KS, jnp.int32)

    def sbody(i, _):
        sl = pl.ds(i * 16, 16)
        lab16 = lab_v[sl].astype(jnp.int32)
        cf16 = cf_v[sl]
        plsc.addupdate_scatter(bins_v, [lab16], ones16)
        plsc.addupdate_scatter(bins_v, [lab16 + koff], cf16)
        plsc.addupdate_scatter(bins_v, [lab16 + koff + koff], cf16 * cf16)
        return 0

    jax.lax.fori_loop(0, _CH // 16, sbody, 0)

    pltpu.sync_copy(bins_v, stage.at[pl.ds(sid * _BINS, _BINS)])
    plsc.subcore_barrier()
    off = sid * _SL
    for r in range(_NSUB):
        pltpu.sync_copy(stage.at[pl.ds(r * _BINS + off, _SL)],
                        acc_v.at[pl.ds(r * _SL, _SL)])
    for i in range(_SL // 16):
        sl = pl.ds(i * 16, 16)
        tot = acc_v[sl]
        for r in range(1, _NSUB):
            tot = tot + acc_v[pl.ds(r * _SL + i * 16, 16)]
        bins_v[pl.ds(i * 16, 16)] = tot
    pltpu.sync_copy(bins_v.at[pl.ds(0, _SL)], total.at[pl.ds(off, _SL)])
    plsc.subcore_barrier()
    pltpu.sync_copy(total, bins_v)

    def mbody(i, _):
        sl = pl.ds(i * 16, 16)
        c = bins_v[sl]
        s = bins_v[pl.ds(_KS + i * 16, 16)]
        q = bins_v[pl.ds(2 * _KS + i * 16, 16)]
        m = s / jnp.maximum(c, 1.0)
        var = (q - c * m * m) / jnp.maximum(c - 1.0, 1.0)
        var = jnp.maximum(var, 1e-12)
        ib = plsc.bitcast(var, jnp.int32)
        y = plsc.bitcast(jnp.int32(0x5F3759DF) - (ib >> 1), jnp.float32)
        y = y * (1.5 - 0.5 * var * y * y)
        y = y * (1.5 - 0.5 * var * y * y)
        y = y * (1.5 - 0.5 * var * y * y)
        mean_v[sl] = m
        isd_v[sl] = y
        return 0

    jax.lax.fori_loop(0, _KS // 16, mbody, 0)

    def wbody(i, _):
        sl = pl.ds(i * 16, 16)
        lab16 = lab_v[sl].astype(jnp.int32)
        cf16 = cf_v[sl]
        m = plsc.load_gather(mean_v, [lab16])
        isd = plsc.load_gather(isd_v, [lab16])
        z = (cf16 - m) * isd
        pdf = jnp.exp(-0.5 * z * z) * isd * _INV_SQRT_2PI
        out_v[sl] = jnp.where(cf16 < m, pdf, ones16)
        return 0

    jax.lax.fori_loop(0, _CH // 16, wbody, 0)
    pltpu.sync_copy(out_v, w_hbm.at[pl.ds(base, _CH)])


_sc_call = functools.partial(
    pl.kernel,
    mesh=plsc.VectorSubcoreMesh(core_axis_name="c", subcore_axis_name="s",
                                num_cores=1),
    compiler_params=pltpu.CompilerParams(needs_layout_passes=False),
    out_type=jax.ShapeDtypeStruct((_B,), jnp.float32),
    scratch_types=[
        pltpu.VMEM((_CH,), jnp.float32),
        pltpu.VMEM((_CH,), jnp.float32),
        pltpu.VMEM((_BINS,), jnp.float32),
        pltpu.VMEM((_KS,), jnp.float32),
        pltpu.VMEM((_KS,), jnp.float32),
        pltpu.VMEM((_CH,), jnp.float32),
        pltpu.VMEM((_NSUB * _SL,), jnp.float32),
        pltpu.VMEM_SHARED((_NSUB * _BINS,), jnp.float32),
        pltpu.VMEM_SHARED((_BINS,), jnp.float32),
    ],
)(_sc_weight)


@functools.partial(jax.jit, static_argnames=())
def kernel(feature, pred, unlabeled_index, centroids):
    del pred, unlabeled_index
    label2d, cosf2d = pl.pallas_call(
        _tc_kernel,
        out_shape=(
            jax.ShapeDtypeStruct((_B, 1), jnp.float32),
            jax.ShapeDtypeStruct((_B, 1), jnp.float32),
        ),
    )(feature, centroids)
    label = label2d.reshape(_B)
    cosf = cosf2d.reshape(_B)
    weight = _sc_call(cosf, label)
    return label, weight


# speedup vs baseline: 1.3416x; 1.0099x over previous
import functools

import jax
import jax.numpy as jnp
from jax.experimental import pallas as pl
from jax.experimental.pallas import tpu as pltpu
from jax.experimental.pallas import tpu_sc as plsc

_B = 16384
_D = 64
_K = 1000
_BS = 8192
_NB = _B // _BS

_NSUB = 16
_CH = _B // _NSUB
_KS = 1024
_BINS = 3 * _KS
_SL = _BINS // _NSUB

_INV_SQRT_2PI = 0.3989422804014327
_CONTRACT_NT = (((1,), (1,)), ((), ()))


def _tc_kernel(feature_ref, cnt_ref, label_ref, cosf_ref):
    eps = 1e-8
    cnt = cnt_ref[...]
    cnorm = jnp.sqrt(jnp.sum(cnt * cnt, axis=1, keepdims=True))
    cnn = cnt / jnp.maximum(cnorm, eps)

    def body(j, _):
        f = feature_ref[pl.ds(j * _BS, _BS), :]
        fnorm = jnp.sqrt(jnp.sum(f * f, axis=1, keepdims=True))
        fn = f / jnp.maximum(fnorm, eps)
        cos = jax.lax.dot_general(fn, cnn, _CONTRACT_NT,
                                  preferred_element_type=jnp.float32)
        cos_f = jnp.max(cos, axis=1, keepdims=True)
        col_iota = jax.lax.broadcasted_iota(jnp.int32, (_BS, _K), 1)
        lab = jnp.min(jnp.where(cos == cos_f, col_iota, _K), axis=1,
                      keepdims=True)
        cosf_ref[pl.ds(j * _BS, _BS), :] = cos_f
        label_ref[pl.ds(j * _BS, _BS), :] = lab.astype(jnp.float32)
        return 0

    jax.lax.fori_loop(0, _NB, body, 0)


def _sc_weight(cosf_hbm, lab_hbm, w_hbm,
               cf_v, lab_v, bins_v, mean_v, isd_v, out_v, acc_v,
               stage, total):
    sid = jax.lax.axis_index("s")
    base = sid * _CH
    pltpu.sync_copy(cosf_hbm.at[pl.ds(base, _CH)], cf_v)
    pltpu.sync_copy(lab_hbm.at[pl.ds(base, _CH)], lab_v)

    zeros16 = jnp.zeros((16,), jnp.float32)
    ones16 = jnp.ones((16,), jnp.float32)

    def zbody(i, _):
        bins_v[pl.ds(i * 16, 16)] = zeros16
        return 0

    jax.lax.fori_loop(0, _BINS // 16, zbody, 0)

    koff = jnp.full((16,), _KS, jnp.int32)

    def sbody(i, _):
        sl = pl.ds(i * 16, 16)
        lab16 = lab_v[sl].astype(jnp.int32)
        cf16 = cf_v[sl]
        plsc.addupdate_scatter(bins_v, [lab16], ones16)
        plsc.addupdate_scatter(bins_v, [lab16 + koff], cf16)
        plsc.addupdate_scatter(bins_v, [lab16 + koff + koff], cf16 * cf16)
        return 0

    jax.lax.fori_loop(0, _CH // 16, sbody, 0)

    pltpu.sync_copy(bins_v, stage.at[pl.ds(sid * _BINS, _BINS)])
    plsc.subcore_barrier()
    off = sid * _SL
    for r in range(_NSUB):
        pltpu.sync_copy(stage.at[pl.ds(r * _BINS + off, _SL)],
                        acc_v.at[pl.ds(r * _SL, _SL)])
    for i in range(_SL // 16):
        sl = pl.ds(i * 16, 16)
        tot = acc_v[sl]
        for r in range(1, _NSUB):
            tot = tot + acc_v[pl.ds(r * _SL + i * 16, 16)]
        bins_v[pl.ds(i * 16, 16)] = tot
    pltpu.sync_copy(bins_v.at[pl.ds(0, _SL)], total.at[pl.ds(off, _SL)])
    plsc.subcore_barrier()
    pltpu.sync_copy(total, bins_v)

    def mbody(i, _):
        sl = pl.ds(i * 16, 16)
        c = bins_v[sl]
        s = bins_v[pl.ds(_KS + i * 16, 16)]
        q = bins_v[pl.ds(2 * _KS + i * 16, 16)]
        m = s / jnp.maximum(c, 1.0)
        var = (q - c * m * m) / jnp.maximum(c - 1.0, 1.0)
        var = jnp.maximum(var, 1e-12)
        ib = plsc.bitcast(var, jnp.int32)
        y = plsc.bitcast(jnp.int32(0x5F3759DF) - (ib >> 1), jnp.float32)
        y = y * (1.5 - 0.5 * var * y * y)
        y = y * (1.5 - 0.5 * var * y * y)
        y = y * (1.5 - 0.5 * var * y * y)
        mean_v[sl] = m
        isd_v[sl] = y
        return 0

    jax.lax.fori_loop(0, _KS // 16, mbody, 0)

    def wbody(i, _):
        sl = pl.ds(i * 16, 16)
        lab16 = lab_v[sl].astype(jnp.int32)
        cf16 = cf_v[sl]
        m = plsc.load_gather(mean_v, [lab16])
        isd = plsc.load_gather(isd_v, [lab16])
        z = (cf16 - m) * isd
        pdf = jnp.exp(-0.5 * z * z) * isd * _INV_SQRT_2PI
        out_v[sl] = jnp.where(cf16 < m, pdf, ones16)
        return 0

    jax.lax.fori_loop(0, _CH // 16, wbody, 0)
    pltpu.sync_copy(out_v, w_hbm.at[pl.ds(base, _CH)])


_sc_call = functools.partial(
    pl.kernel,
    mesh=plsc.VectorSubcoreMesh(core_axis_name="c", subcore_axis_name="s",
                                num_cores=1),
    compiler_params=pltpu.CompilerParams(needs_layout_passes=False),
    out_type=jax.ShapeDtypeStruct((_B,), jnp.float32),
    scratch_types=[
        pltpu.VMEM((_CH,), jnp.float32),
        pltpu.VMEM((_CH,), jnp.float32),
        pltpu.VMEM((_BINS,), jnp.float32),
        pltpu.VMEM((_KS,), jnp.float32),
        pltpu.VMEM((_KS,), jnp.float32),
        pltpu.VMEM((_CH,), jnp.float32),
        pltpu.VMEM((_NSUB * _SL,), jnp.float32),
        pltpu.VMEM_SHARED((_NSUB * _BINS,), jnp.float32),
        pltpu.VMEM_SHARED((_BINS,), jnp.float32),
    ],
)(_sc_weight)


@functools.partial(jax.jit, static_argnames=())
def kernel(feature, pred, unlabeled_index, centroids):
    del pred, unlabeled_index
    label2d, cosf2d = pl.pallas_call(
        _tc_kernel,
        out_shape=(
            jax.ShapeDtypeStruct((_B, 1), jnp.float32),
            jax.ShapeDtypeStruct((_B, 1), jnp.float32),
        ),
    )(feature, centroids)
    label = label2d.reshape(_B)
    cosf = cosf2d.reshape(_B)
    weight = _sc_call(cosf, label)
    return label, weight
